# baseline (speedup 1.0000x reference)
import functools
import math

import jax
import jax.numpy as jnp
from jax.experimental import pallas as pl
from jax.experimental.pallas import tpu as pltpu


def _fused_attn_kernel(x_ref, wqkv_ref, wout_ref, y_ref, *, sm_scale, head_dim):
    h = head_dim
    x = x_ref[...]
    qkv = jnp.dot(x, wqkv_ref[...], preferred_element_type=jnp.float32)
    q = qkv[:, :h]
    k = qkv[:, h:2 * h]
    v = qkv[:, 2 * h:]

    s = jax.lax.dot_general(
        q.astype(jnp.bfloat16), k.astype(jnp.bfloat16),
        (((1,), (1,)), ((), ())), preferred_element_type=jnp.float32,
    ) * sm_scale

    seq = s.shape[0]
    row = jax.lax.broadcasted_iota(jnp.int32, (seq, seq), 0)
    col = jax.lax.broadcasted_iota(jnp.int32, (seq, seq), 1)
    s = s + jnp.where(col <= row, jnp.float32(0.0), jnp.float32(-1e10))

    m = jnp.max(s, axis=-1, keepdims=True)
    p = jnp.exp(s - m)
    l = jnp.sum(p, axis=-1, keepdims=True)
    attn = (p / l).astype(jnp.bfloat16)

    o = jnp.dot(attn, v.astype(jnp.bfloat16),
                preferred_element_type=jnp.float32)

    w_sum = jnp.sum(wout_ref[...], axis=0)
    r = seq // h
    yt = o.reshape(seq // r, r, h).transpose(1, 2, 0).reshape(seq, seq // r)
    y_ref[...] = (yt * w_sum[None, :]).astype(y_ref.dtype)


def kernel(x, w_qkv, w_out):
    batch, seq, d_model = x.shape
    n_heads, d_model_w, three_h = w_qkv.shape
    head_dim = three_h // 3
    assert d_model_w == d_model and n_heads * head_dim == d_model
    assert seq % head_dim == 0

    xb = x.astype(jnp.bfloat16)
    wb = w_qkv.astype(jnp.bfloat16)
    w_flat = w_out.reshape(n_heads * head_dim, d_model)

    cost = pl.CostEstimate(
        flops=2 * batch * n_heads * seq * d_model * 3 * head_dim
        + 4 * batch * n_heads * seq * seq * head_dim,
        transcendentals=batch * n_heads * seq * seq,
        bytes_accessed=2 * batch * seq * d_model
        + 2 * n_heads * d_model * 3 * head_dim
        + 4 * n_heads * head_dim * d_model
        + 4 * batch * seq * d_model,
    )

    y = pl.pallas_call(
        functools.partial(
            _fused_attn_kernel,
            sm_scale=1.0 / math.sqrt(head_dim),
            head_dim=head_dim,
        ),
        out_shape=jax.ShapeDtypeStruct((batch, seq, d_model), x.dtype),
        grid=(batch, n_heads),
        in_specs=[
            pl.BlockSpec((None, seq, d_model), lambda b, n: (b, 0, 0)),
            pl.BlockSpec((None, d_model, three_h), lambda b, n: (n, 0, 0)),
            pl.BlockSpec((n_heads * head_dim, head_dim), lambda b, n: (0, n)),
        ],
        out_specs=pl.BlockSpec((None, seq, head_dim), lambda b, n: (b, 0, n)),
        compiler_params=pltpu.CompilerParams(
            dimension_semantics=("parallel", "parallel")
        ),
        cost_estimate=cost,
    )(xb, wb, w_flat)
    return y


# speedup vs baseline: 2.6208x; 2.6208x over previous
import functools
import math

import jax
import jax.numpy as jnp
from jax.experimental import pallas as pl
from jax.experimental.pallas import tpu as pltpu


def _fused_attn_kernel(x_ref, wqkv_ref, wout_ref, y_ref, *, sm_scale, head_dim, r):
    h = head_dim
    x = x_ref[...]
    qkv = jnp.dot(x, wqkv_ref[...], preferred_element_type=jnp.float32)
    q = qkv[:, :h]
    k = qkv[:, h:2 * h]
    v = qkv[:, 2 * h:]

    s = jax.lax.dot_general(
        q.astype(jnp.bfloat16), k.astype(jnp.bfloat16),
        (((1,), (1,)), ((), ())), preferred_element_type=jnp.float32,
    ) * sm_scale

    seq = s.shape[0]
    blk = seq // r
    ri = jax.lax.broadcasted_iota(jnp.int32, (seq, seq), 0)
    ci = jax.lax.broadcasted_iota(jnp.int32, (seq, seq), 1)
    orow = (ri % blk) * r + ri // blk
    ocol = (ci % blk) * r + ci // blk
    s = jnp.where(ocol <= orow, s, jnp.float32(-1e10))

    m = jnp.max(s, axis=-1, keepdims=True)
    p = jnp.exp(s - m)
    l = jnp.sum(p, axis=-1, keepdims=True)
    attn = (p * (1.0 / l)).astype(jnp.bfloat16)

    vb = v.astype(jnp.bfloat16)
    w_sum = jnp.sum(wout_ref[...], axis=0)[None, :]

    for bp in range(r):
        ab = attn[bp * blk:(bp + 1) * blk, :]
        ytb = jax.lax.dot_general(
            vb, ab, (((0,), (1,)), ((), ())),
            preferred_element_type=jnp.float32,
        )
        y_ref[bp * h:(bp + 1) * h, :] = (ytb * w_sum).astype(y_ref.dtype)


def kernel(x, w_qkv, w_out):
    batch, seq, d_model = x.shape
    n_heads, d_model_w, three_h = w_qkv.shape
    head_dim = three_h // 3
    assert d_model_w == d_model and n_heads * head_dim == d_model
    assert seq % head_dim == 0
    r = seq // head_dim

    xb = (x.astype(jnp.bfloat16)
          .reshape(batch, seq // r, r, d_model)
          .swapaxes(1, 2)
          .reshape(batch, seq, d_model))
    wb = w_qkv.astype(jnp.bfloat16)
    w_flat = w_out.reshape(n_heads * head_dim, d_model)

    cost = pl.CostEstimate(
        flops=2 * batch * n_heads * seq * d_model * 3 * head_dim
        + 4 * batch * n_heads * seq * seq * head_dim,
        transcendentals=batch * n_heads * seq * seq,
        bytes_accessed=2 * batch * seq * d_model
        + 2 * n_heads * d_model * 3 * head_dim
        + 4 * n_heads * head_dim * d_model
        + 4 * batch * seq * d_model,
    )

    y = pl.pallas_call(
        functools.partial(
            _fused_attn_kernel,
            sm_scale=1.0 / math.sqrt(head_dim),
            head_dim=head_dim,
            r=r,
        ),
        out_shape=jax.ShapeDtypeStruct((batch, seq, d_model), x.dtype),
        grid=(batch, n_heads),
        in_specs=[
            pl.BlockSpec((None, seq, d_model), lambda b, n: (b, 0, 0)),
            pl.BlockSpec((None, d_model, three_h), lambda b, n: (n, 0, 0)),
            pl.BlockSpec((n_heads * head_dim, head_dim), lambda b, n: (0, n)),
        ],
        out_specs=pl.BlockSpec((None, seq, head_dim), lambda b, n: (b, 0, n)),
        compiler_params=pltpu.CompilerParams(
            dimension_semantics=("parallel", "parallel")
        ),
        cost_estimate=cost,
    )(xb, wb, w_flat)
    return y
